# baseline (speedup 1.0000x reference)
import jax
import jax.numpy as jnp
from jax.experimental import pallas as pl
from jax.experimental.pallas import tpu as pltpu

_B = 32
_T = 576
_LATENT = 256
_NUM_EMB = 1024
_HID = 1024
_OUT = 768
_CC = 0.25

_ROWS = _B * _T
_BLK = 3072
_GRID = _ROWS // _BLK
_HALF = _BLK // 2


def _dot_t(x, w):
    return jax.lax.dot_general(x, w, (((1,), (1,)), ((), ())),
                               preferred_element_type=jnp.float32)


def _fused_body(z_ref, embm2_ref, emb_ref, esq_ref,
                w0_ref, b0_ref, w1_ref, b1_ref, w2_ref, b2_ref,
                recon_ref, qst_ref, idx_ref, loss_ref):
    def quantize(r0):
        sl = pl.ds(r0, _HALF)
        zb = z_ref[sl, :]
        scores = _dot_t(zb, embm2_ref[...])
        zsq = jnp.sum(zb * zb, axis=1, keepdims=True)
        dist = (zsq + scores) + esq_ref[...]
        dmin = jnp.min(dist, axis=1, keepdims=True)
        lane = jax.lax.broadcasted_iota(jnp.int32, dist.shape, 1)
        idx = jnp.min(jnp.where(dist == dmin, lane, _NUM_EMB), axis=1)
        idx = idx.astype(jnp.int32)
        idx_ref[0, 0, sl] = idx
        oh = (lane == idx[:, None]).astype(jnp.float32)
        q = jnp.dot(oh, emb_ref[...],
                    preferred_element_type=jnp.float32)
        qst = zb + (q - zb)
        qst_ref[sl, :] = qst
        diff = zb - q
        return qst, jnp.sum(diff * diff)

    def decode(r0, qst):
        sl = pl.ds(r0, _HALF)
        h = jnp.tanh(_dot_t(qst, w0_ref[...]) + b0_ref[...])
        h = jnp.tanh(_dot_t(h, w1_ref[...]) + b1_ref[...])
        recon_ref[sl, :] = _dot_t(h, w2_ref[...]) + b2_ref[...]

    qst_a, loss_a = quantize(0)
    qst_b, loss_b = quantize(_HALF)
    decode(0, qst_a)
    decode(_HALF, qst_b)
    loss_ref[...] = (loss_a + loss_b).reshape(1, 1, 1)


def kernel(state, z, embeddings, W0, b0, W1, b1, W2, b2):
    del state
    flat_z = z.reshape(_ROWS, _LATENT)
    e_sq = jnp.sum(embeddings ** 2, axis=1).reshape(1, _NUM_EMB)
    embm2 = (-2.0) * embeddings
    b0r = b0.reshape(1, _HID)
    b1r = b1.reshape(1, _HID)
    b2r = b2.reshape(1, _OUT)

    full = lambda shape: pl.BlockSpec(shape, lambda i: (0,) * len(shape))
    recon, qst, idx3, loss_parts = pl.pallas_call(
        _fused_body,
        grid=(_GRID,),
        in_specs=[
            pl.BlockSpec((_BLK, _LATENT), lambda i: (i, 0)),
            full((_NUM_EMB, _LATENT)),
            full((_NUM_EMB, _LATENT)),
            full((1, _NUM_EMB)),
            full((_HID, _LATENT)),
            full((1, _HID)),
            full((_HID, _HID)),
            full((1, _HID)),
            full((_OUT, _HID)),
            full((1, _OUT)),
        ],
        out_specs=[
            pl.BlockSpec((_BLK, _OUT), lambda i: (i, 0)),
            pl.BlockSpec((_BLK, _LATENT), lambda i: (i, 0)),
            pl.BlockSpec((1, 1, _BLK), lambda i: (i, 0, 0)),
            pl.BlockSpec((1, 1, 1), lambda i: (i, 0, 0)),
        ],
        out_shape=[
            jax.ShapeDtypeStruct((_ROWS, _OUT), jnp.float32),
            jax.ShapeDtypeStruct((_ROWS, _LATENT), jnp.float32),
            jax.ShapeDtypeStruct((_GRID, 1, _BLK), jnp.int32),
            jax.ShapeDtypeStruct((_GRID, 1, 1), jnp.float32),
        ],
        compiler_params=pltpu.CompilerParams(
            dimension_semantics=("parallel",)),
    )(flat_z, embm2, embeddings, e_sq,
      W0, b0r, W1, b1r, W2, b2r)

    recon = recon.reshape(_B, _T, _OUT)
    quantized_st = qst.reshape(_B, _T, _LATENT)
    indices = idx3.reshape(_B, _T)
    vq_loss = (jnp.sum(loss_parts) * ((1.0 + _CC) / (_ROWS * _LATENT))).astype(jnp.float32)
    return recon, quantized_st, vq_loss, indices


# speedup vs baseline: 1.0145x; 1.0145x over previous
import jax
import jax.numpy as jnp
from jax.experimental import pallas as pl
from jax.experimental.pallas import tpu as pltpu

_B = 32
_T = 576
_LATENT = 256
_NUM_EMB = 1024
_HID = 1024
_OUT = 768
_CC = 0.25

_ROWS = _B * _T
_BLK = 2048
_GRID = _ROWS // _BLK
_NSPLIT = 4
_HALF = _BLK // _NSPLIT


def _dot_t(x, w):
    return jax.lax.dot_general(x, w, (((1,), (1,)), ((), ())),
                               preferred_element_type=jnp.float32)


def _fused_body(z_ref, embm2_ref, emb_ref, esq_ref,
                w0_ref, b0_ref, w1_ref, b1_ref, w2_ref, b2_ref,
                recon_ref, qst_ref, idx_ref, loss_ref):
    def quantize(r0):
        sl = pl.ds(r0, _HALF)
        zb = z_ref[sl, :]
        scores = _dot_t(zb, embm2_ref[...])
        zsq = jnp.sum(zb * zb, axis=1, keepdims=True)
        dist = (zsq + scores) + esq_ref[...]
        dmin = jnp.min(dist, axis=1, keepdims=True)
        lane = jax.lax.broadcasted_iota(jnp.int32, dist.shape, 1)
        idx = jnp.min(jnp.where(dist == dmin, lane, _NUM_EMB), axis=1)
        idx = idx.astype(jnp.int32)
        idx_ref[0, 0, sl] = idx
        oh = (lane == idx[:, None]).astype(jnp.float32)
        q = jnp.dot(oh, emb_ref[...],
                    preferred_element_type=jnp.float32)
        qst = zb + (q - zb)
        qst_ref[sl, :] = qst
        diff = zb - q
        return qst, jnp.sum(diff * diff)

    def decode(r0, qst):
        sl = pl.ds(r0, _HALF)
        h = jnp.tanh(_dot_t(qst, w0_ref[...]) + b0_ref[...])
        h = jnp.tanh(_dot_t(h, w1_ref[...]) + b1_ref[...])
        recon_ref[sl, :] = _dot_t(h, w2_ref[...]) + b2_ref[...]

    parts = [quantize(k * _HALF) for k in range(_NSPLIT)]
    for k, (qst_k, _) in enumerate(parts):
        decode(k * _HALF, qst_k)
    total = parts[0][1]
    for _, loss_k in parts[1:]:
        total = total + loss_k
    loss_ref[...] = total.reshape(1, 1, 1)


def kernel(state, z, embeddings, W0, b0, W1, b1, W2, b2):
    del state
    flat_z = z.reshape(_ROWS, _LATENT)
    e_sq = jnp.sum(embeddings ** 2, axis=1).reshape(1, _NUM_EMB)
    embm2 = (-2.0) * embeddings
    b0r = b0.reshape(1, _HID)
    b1r = b1.reshape(1, _HID)
    b2r = b2.reshape(1, _OUT)

    full = lambda shape: pl.BlockSpec(shape, lambda i: (0,) * len(shape))
    recon, qst, idx3, loss_parts = pl.pallas_call(
        _fused_body,
        grid=(_GRID,),
        in_specs=[
            pl.BlockSpec((_BLK, _LATENT), lambda i: (i, 0)),
            full((_NUM_EMB, _LATENT)),
            full((_NUM_EMB, _LATENT)),
            full((1, _NUM_EMB)),
            full((_HID, _LATENT)),
            full((1, _HID)),
            full((_HID, _HID)),
            full((1, _HID)),
            full((_OUT, _HID)),
            full((1, _OUT)),
        ],
        out_specs=[
            pl.BlockSpec((_BLK, _OUT), lambda i: (i, 0)),
            pl.BlockSpec((_BLK, _LATENT), lambda i: (i, 0)),
            pl.BlockSpec((1, 1, _BLK), lambda i: (i, 0, 0)),
            pl.BlockSpec((1, 1, 1), lambda i: (i, 0, 0)),
        ],
        out_shape=[
            jax.ShapeDtypeStruct((_ROWS, _OUT), jnp.float32),
            jax.ShapeDtypeStruct((_ROWS, _LATENT), jnp.float32),
            jax.ShapeDtypeStruct((_GRID, 1, _BLK), jnp.int32),
            jax.ShapeDtypeStruct((_GRID, 1, 1), jnp.float32),
        ],
        compiler_params=pltpu.CompilerParams(
            dimension_semantics=("parallel",)),
    )(flat_z, embm2, embeddings, e_sq,
      W0, b0r, W1, b1r, W2, b2r)

    recon = recon.reshape(_B, _T, _OUT)
    quantized_st = qst.reshape(_B, _T, _LATENT)
    indices = idx3.reshape(_B, _T)
    vq_loss = (jnp.sum(loss_parts) * ((1.0 + _CC) / (_ROWS * _LATENT))).astype(jnp.float32)
    return recon, quantized_st, vq_loss, indices


# speedup vs baseline: 1.0421x; 1.0272x over previous
import jax
import jax.numpy as jnp
from jax.experimental import pallas as pl
from jax.experimental.pallas import tpu as pltpu

_B = 32
_T = 576
_LATENT = 256
_NUM_EMB = 1024
_HID = 1024
_OUT = 768
_CC = 0.25

_ROWS = _B * _T
_BLK = 2048
_GRID = _ROWS // _BLK
_NSPLIT = 4
_HALF = _BLK // _NSPLIT


def _dot_t(x, w):
    return jax.lax.dot_general(x, w, (((1,), (1,)), ((), ())),
                               preferred_element_type=jnp.float32)


def _fused_body(z_ref, embm2_ref, emb_ref, esq_ref,
                w0_ref, b0_ref, w1_ref, b1_ref, w2_ref, b2_ref,
                recon_ref, qst_ref, idx_ref, loss_ref):
    def quantize(r0):
        sl = pl.ds(r0, _HALF)
        zb = z_ref[sl, :]
        scores = _dot_t(zb, embm2_ref[...])
        zsq = jnp.sum(zb * zb, axis=1, keepdims=True)
        dist = (zsq + scores) + esq_ref[...]
        dmin = jnp.min(dist, axis=1, keepdims=True)
        lane = jax.lax.broadcasted_iota(jnp.int32, dist.shape, 1)
        idx = jnp.min(jnp.where(dist == dmin, lane, _NUM_EMB), axis=1)
        idx = idx.astype(jnp.int32)
        idx_ref[0, 0, sl] = idx
        oh = (lane == idx[:, None]).astype(jnp.float32)
        q = jnp.dot(oh, emb_ref[...],
                    preferred_element_type=jnp.float32)
        qst = zb + (q - zb)
        qst_ref[sl, :] = qst
        diff = zb - q
        return qst, jnp.sum(diff * diff)

    def decode(r0, qst):
        sl = pl.ds(r0, _HALF)
        h = jnp.tanh(_dot_t(qst, w0_ref[...]) + b0_ref[...])
        h = jnp.tanh(_dot_t(h, w1_ref[...]) + b1_ref[...])
        recon_ref[sl, :] = _dot_t(h, w2_ref[...]) + b2_ref[...]

    parts = [quantize(0), quantize(_HALF)]
    for k in range(_NSPLIT):
        if k + 2 < _NSPLIT:
            parts.append(quantize((k + 2) * _HALF))
        decode(k * _HALF, parts[k][0])
    total = parts[0][1]
    for _, loss_k in parts[1:]:
        total = total + loss_k
    loss_ref[...] = total.reshape(1, 1, 1)


def kernel(state, z, embeddings, W0, b0, W1, b1, W2, b2):
    del state
    flat_z = z.reshape(_ROWS, _LATENT)
    e_sq = jnp.sum(embeddings ** 2, axis=1).reshape(1, _NUM_EMB)
    embm2 = (-2.0) * embeddings
    b0r = b0.reshape(1, _HID)
    b1r = b1.reshape(1, _HID)
    b2r = b2.reshape(1, _OUT)

    full = lambda shape: pl.BlockSpec(shape, lambda i: (0,) * len(shape))
    recon, qst, idx3, loss_parts = pl.pallas_call(
        _fused_body,
        grid=(_GRID,),
        in_specs=[
            pl.BlockSpec((_BLK, _LATENT), lambda i: (i, 0)),
            full((_NUM_EMB, _LATENT)),
            full((_NUM_EMB, _LATENT)),
            full((1, _NUM_EMB)),
            full((_HID, _LATENT)),
            full((1, _HID)),
            full((_HID, _HID)),
            full((1, _HID)),
            full((_OUT, _HID)),
            full((1, _OUT)),
        ],
        out_specs=[
            pl.BlockSpec((_BLK, _OUT), lambda i: (i, 0)),
            pl.BlockSpec((_BLK, _LATENT), lambda i: (i, 0)),
            pl.BlockSpec((1, 1, _BLK), lambda i: (i, 0, 0)),
            pl.BlockSpec((1, 1, 1), lambda i: (i, 0, 0)),
        ],
        out_shape=[
            jax.ShapeDtypeStruct((_ROWS, _OUT), jnp.float32),
            jax.ShapeDtypeStruct((_ROWS, _LATENT), jnp.float32),
            jax.ShapeDtypeStruct((_GRID, 1, _BLK), jnp.int32),
            jax.ShapeDtypeStruct((_GRID, 1, 1), jnp.float32),
        ],
        compiler_params=pltpu.CompilerParams(
            dimension_semantics=("parallel",)),
    )(flat_z, embm2, embeddings, e_sq,
      W0, b0r, W1, b1r, W2, b2r)

    recon = recon.reshape(_B, _T, _OUT)
    quantized_st = qst.reshape(_B, _T, _LATENT)
    indices = idx3.reshape(_B, _T)
    vq_loss = (jnp.sum(loss_parts) * ((1.0 + _CC) / (_ROWS * _LATENT))).astype(jnp.float32)
    return recon, quantized_st, vq_loss, indices
